# baseline (speedup 1.0000x reference)
import functools

import jax
import jax.numpy as jnp
from jax import lax
from jax.experimental import pallas as pl
from jax.experimental.pallas import tpu as pltpu
from jax.experimental.pallas import tpu_sc as plsc

_N = 10000
_E = 160000
_D = 128
_H = 128
_NP = 10240
_EP = 163840
_EROWS = _EP // 128
_ROWS_PER_TILE = _EROWS // 32
_NODE_ROWS_PER_TILE = _NP // 16


def _deg_body(dst_hbm, zeros_hbm, ones_hbm, out_hbm, ones_v, idx_v, acc_sh):
    c = lax.axis_index("c")
    s = lax.axis_index("s")
    nbase = s * _NODE_ROWS_PER_TILE
    pltpu.sync_copy(zeros_hbm.at[pl.ds(nbase, _NODE_ROWS_PER_TILE)],
                    acc_sh.at[pl.ds(nbase, _NODE_ROWS_PER_TILE)])
    pltpu.sync_copy(ones_hbm, ones_v)
    row0 = (c * 16 + s) * _ROWS_PER_TILE
    pltpu.sync_copy(dst_hbm.at[pl.ds(row0, _ROWS_PER_TILE)], idx_v)
    plsc.subcore_barrier()

    def body(i, carry):
        pltpu.sync_copy(ones_v, acc_sh.at[idx_v.at[i]], add=True)
        return carry

    lax.fori_loop(0, _ROWS_PER_TILE, body, 0)
    plsc.subcore_barrier()
    pltpu.sync_copy(acc_sh.at[pl.ds(nbase, _NODE_ROWS_PER_TILE)],
                    out_hbm.at[c, pl.ds(nbase, _NODE_ROWS_PER_TILE)])


def _scatter_body(src_hbm, dst_hbm, xs_hbm, zeros_hbm, out_hbm,
                  sidx, didx, b0, b1, acc_sh, gs0, gs1):
    c = lax.axis_index("c")
    s = lax.axis_index("s")
    nbase = s * _NODE_ROWS_PER_TILE
    pltpu.sync_copy(zeros_hbm.at[pl.ds(nbase, _NODE_ROWS_PER_TILE)],
                    acc_sh.at[pl.ds(nbase, _NODE_ROWS_PER_TILE)])
    row0 = (c * 16 + s) * _ROWS_PER_TILE
    pltpu.sync_copy(src_hbm.at[pl.ds(row0, _ROWS_PER_TILE)], sidx)
    pltpu.sync_copy(dst_hbm.at[pl.ds(row0, _ROWS_PER_TILE)], didx)
    plsc.subcore_barrier()

    bufs = (b0, b1)
    gsems = (gs0, gs1)

    def drain(slot):
        pltpu.make_async_copy(xs_hbm.at[pl.ds(0, 128)], bufs[slot],
                              gsems[slot]).wait()

    def step(ch, slot, issue_next):
        drain(slot)
        if issue_next:
            pltpu.async_copy(xs_hbm.at[sidx.at[ch + 1]], bufs[1 - slot],
                             gsems[1 - slot])
        pltpu.sync_copy(bufs[slot], acc_sh.at[didx.at[ch]], add=True)

    pltpu.async_copy(xs_hbm.at[sidx.at[0]], b0, gs0)

    def body(k, carry):
        step(2 * k, 0, True)
        step(2 * k + 1, 1, True)
        return carry

    lax.fori_loop(0, _ROWS_PER_TILE // 2 - 1, body, 0)
    step(_ROWS_PER_TILE - 2, 0, True)
    step(_ROWS_PER_TILE - 1, 1, False)
    plsc.subcore_barrier()
    pltpu.sync_copy(acc_sh.at[pl.ds(nbase, _NODE_ROWS_PER_TILE)],
                    out_hbm.at[c, pl.ds(nbase, _NODE_ROWS_PER_TILE)])


def _xs_body(x_ref, w_ref, dp_ref, o_ref):
    xw = jnp.dot(x_ref[...], w_ref[...], preferred_element_type=jnp.float32)
    deg = dp_ref[0] + dp_ref[1] + 1.0
    o_ref[...] = xw * lax.rsqrt(deg)


def _h_body(acc_ref, xs_ref, dp_ref, b_ref, h_ref, s1_ref, s2_ref):
    deg = dp_ref[0] + dp_ref[1] + 1.0
    dinv = lax.rsqrt(deg)
    h = (acc_ref[0] + acc_ref[1] + xs_ref[...]) * dinv + b_ref[...]
    h_ref[...] = h
    i = pl.program_id(0)
    rowid = lax.broadcasted_iota(jnp.int32, (1024, 1), 0) + i * 1024
    hm = jnp.where(rowid < _N, h, 0.0)
    s1 = jnp.sum(hm, axis=0, keepdims=True)
    s2 = jnp.sum(hm * hm, axis=0, keepdims=True)
    s1_ref[...] = jnp.broadcast_to(s1, (8, 128))[None]
    s2_ref[...] = jnp.broadcast_to(s2, (8, 128))[None]


def _ab_body(s1_ref, s2_ref, g_ref, be_ref, ab_ref):
    mean = jnp.sum(s1_ref[...][:, 0, :], axis=0, keepdims=True) * (1.0 / _N)
    ex2 = jnp.sum(s2_ref[...][:, 0, :], axis=0, keepdims=True) * (1.0 / _N)
    var = ex2 - mean * mean
    rstd = lax.rsqrt(var + 1e-5)
    a = rstd * g_ref[...]
    cs = be_ref[...] - mean * a
    ab_ref[0] = jnp.broadcast_to(a, (8, 128))
    ab_ref[1] = jnp.broadcast_to(cs, (8, 128))


def _z_body(h_ref, ab_ref, z_ref):
    a = ab_ref[0, 0:1, :]
    cs = ab_ref[1, 0:1, :]
    z_ref[...] = jnp.maximum(h_ref[...] * a + cs, 0.0)


def _dec_body(zi_ref, zj_ref, o_ref):
    zj = zj_ref[pl.ds(0, _N), :]
    o_ref[...] = lax.dot_general(
        zi_ref[...], zj, (((1,), (1,)), ((), ())),
        preferred_element_type=jnp.float32)


def kernel(x, edge_index, W, b, gamma, beta):
    f32 = jnp.float32
    src = edge_index[0]
    dst = edge_index[1]
    pad = _N + jnp.arange(_EP - _E, dtype=jnp.int32) % (_NP - _N)
    src2d = jnp.concatenate([src, pad]).reshape(_EROWS, 128)
    dst2d = jnp.concatenate([dst, pad]).reshape(_EROWS, 128)
    x_pad = jnp.zeros((_NP, _D), f32).at[:_N, :].set(x)
    zerosH = jnp.zeros((_NP, _H), f32)
    onesH = jnp.ones((128, _H), f32)

    mesh = plsc.VectorSubcoreMesh(core_axis_name="c", subcore_axis_name="s")

    deg_kernel = functools.partial(
        pl.kernel, mesh=mesh,
        out_type=jax.ShapeDtypeStruct((2, _NP, _H), f32),
        scratch_types=[
            pltpu.VMEM((128, _H), f32),
            pltpu.VMEM((_ROWS_PER_TILE, 128), jnp.int32),
            pltpu.VMEM_SHARED((_NP, _H), f32),
        ],
    )(_deg_body)
    degparts = deg_kernel(dst2d, zerosH, onesH)[:, :, 0:1]

    xs = pl.pallas_call(
        _xs_body,
        grid=(10,),
        in_specs=[
            pl.BlockSpec((1024, _D), lambda i: (i, 0)),
            pl.BlockSpec((_D, _H), lambda i: (0, 0)),
            pl.BlockSpec((2, 1024, 1), lambda i: (0, i, 0)),
        ],
        out_specs=pl.BlockSpec((1024, _H), lambda i: (i, 0)),
        out_shape=jax.ShapeDtypeStruct((_NP, _H), f32),
    )(x_pad, W, degparts)

    scatter_kernel = functools.partial(
        pl.kernel, mesh=mesh,
        out_type=jax.ShapeDtypeStruct((2, _NP, _H), f32),
        scratch_types=[
            pltpu.VMEM((_ROWS_PER_TILE, 128), jnp.int32),
            pltpu.VMEM((_ROWS_PER_TILE, 128), jnp.int32),
            pltpu.VMEM((128, _H), f32),
            pltpu.VMEM((128, _H), f32),
            pltpu.VMEM_SHARED((_NP, _H), f32),
            pltpu.SemaphoreType.DMA,
            pltpu.SemaphoreType.DMA,
        ],
    )(_scatter_body)
    accparts = scatter_kernel(src2d, dst2d, xs, zerosH)

    h, ps1, ps2 = pl.pallas_call(
        _h_body,
        grid=(10,),
        in_specs=[
            pl.BlockSpec((2, 1024, _H), lambda i: (0, i, 0)),
            pl.BlockSpec((1024, _H), lambda i: (i, 0)),
            pl.BlockSpec((2, 1024, 1), lambda i: (0, i, 0)),
            pl.BlockSpec((_H,), lambda i: (0,)),
        ],
        out_specs=[
            pl.BlockSpec((1024, _H), lambda i: (i, 0)),
            pl.BlockSpec((1, 8, 128), lambda i: (i, 0, 0)),
            pl.BlockSpec((1, 8, 128), lambda i: (i, 0, 0)),
        ],
        out_shape=[
            jax.ShapeDtypeStruct((_NP, _H), f32),
            jax.ShapeDtypeStruct((10, 8, 128), f32),
            jax.ShapeDtypeStruct((10, 8, 128), f32),
        ],
    )(accparts, xs, degparts, b)

    ab = pl.pallas_call(
        _ab_body,
        in_specs=[
            pl.BlockSpec((10, 8, 128), lambda: (0, 0, 0)),
            pl.BlockSpec((10, 8, 128), lambda: (0, 0, 0)),
            pl.BlockSpec((_H,), lambda: (0,)),
            pl.BlockSpec((_H,), lambda: (0,)),
        ],
        out_specs=pl.BlockSpec((2, 8, 128), lambda: (0, 0, 0)),
        out_shape=jax.ShapeDtypeStruct((2, 8, 128), f32),
    )(ps1, ps2, gamma, beta)

    z = pl.pallas_call(
        _z_body,
        grid=(10,),
        in_specs=[
            pl.BlockSpec((1024, _H), lambda i: (i, 0)),
            pl.BlockSpec((2, 8, 128), lambda i: (0, 0, 0)),
        ],
        out_specs=pl.BlockSpec((1024, _H), lambda i: (i, 0)),
        out_shape=jax.ShapeDtypeStruct((_NP, _H), f32),
    )(h, ab)

    adj = pl.pallas_call(
        _dec_body,
        grid=(25,),
        in_specs=[
            pl.BlockSpec((400, _H), lambda i: (i, 0)),
            pl.BlockSpec((_NP, _H), lambda i: (0, 0)),
        ],
        out_specs=pl.BlockSpec((400, _N), lambda i: (i, 0)),
        out_shape=jax.ShapeDtypeStruct((_N, _N), f32),
    )(z, z)
    return adj


# speedup vs baseline: 1.0313x; 1.0313x over previous
import functools

import jax
import jax.numpy as jnp
from jax import lax
from jax.experimental import pallas as pl
from jax.experimental.pallas import tpu as pltpu
from jax.experimental.pallas import tpu_sc as plsc

_N = 10000
_E = 160000
_D = 128
_H = 128
_NP = 10240
_EP = 163840
_EROWS = _EP // 128
_ROWS_PER_TILE = _EROWS // 32
_NODE_ROWS_PER_TILE = _NP // 16


def _deg_body(dst_hbm, zeros_hbm, ones_hbm, out_hbm, ones_v, idx_v, acc_sh):
    c = lax.axis_index("c")
    s = lax.axis_index("s")
    nbase = s * _NODE_ROWS_PER_TILE
    pltpu.sync_copy(zeros_hbm.at[pl.ds(nbase, _NODE_ROWS_PER_TILE)],
                    acc_sh.at[pl.ds(nbase, _NODE_ROWS_PER_TILE)])
    pltpu.sync_copy(ones_hbm, ones_v)
    row0 = (c * 16 + s) * _ROWS_PER_TILE
    pltpu.sync_copy(dst_hbm.at[pl.ds(row0, _ROWS_PER_TILE)], idx_v)
    plsc.subcore_barrier()

    def body(i, carry):
        pltpu.sync_copy(ones_v, acc_sh.at[idx_v.at[i]], add=True)
        return carry

    lax.fori_loop(0, _ROWS_PER_TILE, body, 0)
    plsc.subcore_barrier()
    pltpu.sync_copy(acc_sh.at[pl.ds(nbase, _NODE_ROWS_PER_TILE)],
                    out_hbm.at[c, pl.ds(nbase, _NODE_ROWS_PER_TILE)])


def _scatter_body(src_hbm, dst_hbm, xs_hbm, zeros_hbm, out_hbm,
                  sidx, didx, b0, b1, acc_sh, gs0, gs1):
    c = lax.axis_index("c")
    s = lax.axis_index("s")
    nbase = s * _NODE_ROWS_PER_TILE
    pltpu.sync_copy(zeros_hbm.at[pl.ds(nbase, _NODE_ROWS_PER_TILE)],
                    acc_sh.at[pl.ds(nbase, _NODE_ROWS_PER_TILE)])
    row0 = (c * 16 + s) * _ROWS_PER_TILE
    pltpu.sync_copy(src_hbm.at[pl.ds(row0, _ROWS_PER_TILE)], sidx)
    pltpu.sync_copy(dst_hbm.at[pl.ds(row0, _ROWS_PER_TILE)], didx)
    plsc.subcore_barrier()

    bufs = (b0, b1)
    gsems = (gs0, gs1)

    def drain(slot):
        pltpu.make_async_copy(xs_hbm.at[pl.ds(0, 128)], bufs[slot],
                              gsems[slot]).wait()

    def step(ch, slot, issue_next):
        drain(slot)
        if issue_next:
            pltpu.async_copy(xs_hbm.at[sidx.at[ch + 1]], bufs[1 - slot],
                             gsems[1 - slot])
        pltpu.sync_copy(bufs[slot], acc_sh.at[didx.at[ch]], add=True)

    pltpu.async_copy(xs_hbm.at[sidx.at[0]], b0, gs0)

    def body(k, carry):
        step(2 * k, 0, True)
        step(2 * k + 1, 1, True)
        return carry

    lax.fori_loop(0, _ROWS_PER_TILE // 2 - 1, body, 0)
    step(_ROWS_PER_TILE - 2, 0, True)
    step(_ROWS_PER_TILE - 1, 1, False)
    plsc.subcore_barrier()
    pltpu.sync_copy(acc_sh.at[pl.ds(nbase, _NODE_ROWS_PER_TILE)],
                    out_hbm.at[c, pl.ds(nbase, _NODE_ROWS_PER_TILE)])


def _xs_body(x_ref, w_ref, dp_ref, o_ref):
    xw = jnp.dot(x_ref[...], w_ref[...], preferred_element_type=jnp.float32)
    deg = dp_ref[0] + dp_ref[1] + 1.0
    o_ref[...] = xw * lax.rsqrt(deg)


def _bn_body(acc_ref, xs_ref, dp_ref, b_ref, g_ref, be_ref, z_ref):
    deg = dp_ref[0] + dp_ref[1] + 1.0
    dinv = lax.rsqrt(deg)
    h = (acc_ref[0] + acc_ref[1] + xs_ref[...]) * dinv + b_ref[...]
    rowid = lax.broadcasted_iota(jnp.int32, (_NP, 1), 0)
    m = rowid < _N
    hm = jnp.where(m, h, 0.0)
    mean = jnp.sum(hm, axis=0, keepdims=True) * (1.0 / _N)
    ex2 = jnp.sum(hm * hm, axis=0, keepdims=True) * (1.0 / _N)
    var = ex2 - mean * mean
    rstd = lax.rsqrt(var + 1e-5)
    zn = (h - mean) * rstd * g_ref[...] + be_ref[...]
    z_ref[...] = jnp.where(m, jnp.maximum(zn, 0.0), 0.0)


def _dec_body(zi_ref, zj_ref, o_ref):
    zj = zj_ref[pl.ds(0, _N), :]
    o_ref[...] = lax.dot_general(
        zi_ref[...], zj, (((1,), (1,)), ((), ())),
        preferred_element_type=jnp.float32)


def kernel(x, edge_index, W, b, gamma, beta):
    f32 = jnp.float32
    src = edge_index[0]
    dst = edge_index[1]
    pad = _N + jnp.arange(_EP - _E, dtype=jnp.int32) % (_NP - _N)
    src2d = jnp.concatenate([src, pad]).reshape(_EROWS, 128)
    dst2d = jnp.concatenate([dst, pad]).reshape(_EROWS, 128)
    x_pad = jnp.zeros((_NP, _D), f32).at[:_N, :].set(x)
    zerosH = jnp.zeros((_NP, _H), f32)
    onesH = jnp.ones((128, _H), f32)

    mesh = plsc.VectorSubcoreMesh(core_axis_name="c", subcore_axis_name="s")

    deg_kernel = functools.partial(
        pl.kernel, mesh=mesh,
        out_type=jax.ShapeDtypeStruct((2, _NP, _H), f32),
        scratch_types=[
            pltpu.VMEM((128, _H), f32),
            pltpu.VMEM((_ROWS_PER_TILE, 128), jnp.int32),
            pltpu.VMEM_SHARED((_NP, _H), f32),
        ],
    )(_deg_body)
    degparts = deg_kernel(dst2d, zerosH, onesH)[:, :, 0:1]

    xs = pl.pallas_call(
        _xs_body,
        grid=(10,),
        in_specs=[
            pl.BlockSpec((1024, _D), lambda i: (i, 0)),
            pl.BlockSpec((_D, _H), lambda i: (0, 0)),
            pl.BlockSpec((2, 1024, 1), lambda i: (0, i, 0)),
        ],
        out_specs=pl.BlockSpec((1024, _H), lambda i: (i, 0)),
        out_shape=jax.ShapeDtypeStruct((_NP, _H), f32),
    )(x_pad, W, degparts)

    scatter_kernel = functools.partial(
        pl.kernel, mesh=mesh,
        out_type=jax.ShapeDtypeStruct((2, _NP, _H), f32),
        scratch_types=[
            pltpu.VMEM((_ROWS_PER_TILE, 128), jnp.int32),
            pltpu.VMEM((_ROWS_PER_TILE, 128), jnp.int32),
            pltpu.VMEM((128, _H), f32),
            pltpu.VMEM((128, _H), f32),
            pltpu.VMEM_SHARED((_NP, _H), f32),
            pltpu.SemaphoreType.DMA,
            pltpu.SemaphoreType.DMA,
        ],
    )(_scatter_body)
    accparts = scatter_kernel(src2d, dst2d, xs, zerosH)

    z = pl.pallas_call(
        _bn_body,
        in_specs=[
            pl.BlockSpec((2, _NP, _H), lambda: (0, 0, 0)),
            pl.BlockSpec((_NP, _H), lambda: (0, 0)),
            pl.BlockSpec((2, _NP, 1), lambda: (0, 0, 0)),
            pl.BlockSpec((_H,), lambda: (0,)),
            pl.BlockSpec((_H,), lambda: (0,)),
            pl.BlockSpec((_H,), lambda: (0,)),
        ],
        out_specs=pl.BlockSpec((_NP, _H), lambda: (0, 0)),
        out_shape=jax.ShapeDtypeStruct((_NP, _H), f32),
    )(accparts, xs, degparts, b, gamma, beta)

    adj = pl.pallas_call(
        _dec_body,
        grid=(25,),
        in_specs=[
            pl.BlockSpec((400, _H), lambda i: (i, 0)),
            pl.BlockSpec((_NP, _H), lambda i: (0, 0)),
        ],
        out_specs=pl.BlockSpec((400, _N), lambda i: (i, 0)),
        out_shape=jax.ShapeDtypeStruct((_N, _N), f32),
    )(z, z)
    return adj


# speedup vs baseline: 1.0355x; 1.0041x over previous
import functools

import jax
import jax.numpy as jnp
from jax import lax
from jax.experimental import pallas as pl
from jax.experimental.pallas import tpu as pltpu
from jax.experimental.pallas import tpu_sc as plsc

_N = 10000
_E = 160000
_D = 128
_H = 128
_NP = 10240
_EP = 163840
_EROWS = _EP // 128
_ROWS_PER_TILE = _EROWS // 32
_NODE_ROWS_PER_TILE = _NP // 16


def _deg_body(dst_hbm, zeros_hbm, ones_hbm, out_hbm, ones_v, idx_v, acc_sh):
    c = lax.axis_index("c")
    s = lax.axis_index("s")
    nbase = s * _NODE_ROWS_PER_TILE
    pltpu.sync_copy(zeros_hbm.at[pl.ds(nbase, _NODE_ROWS_PER_TILE)],
                    acc_sh.at[pl.ds(nbase, _NODE_ROWS_PER_TILE)])
    pltpu.sync_copy(ones_hbm, ones_v)
    row0 = (c * 16 + s) * _ROWS_PER_TILE
    pltpu.sync_copy(dst_hbm.at[pl.ds(row0, _ROWS_PER_TILE)], idx_v)
    plsc.subcore_barrier()

    def body(i, carry):
        pltpu.sync_copy(ones_v, acc_sh.at[idx_v.at[i]], add=True)
        return carry

    lax.fori_loop(0, _ROWS_PER_TILE, body, 0)
    plsc.subcore_barrier()
    pltpu.sync_copy(acc_sh.at[pl.ds(nbase, _NODE_ROWS_PER_TILE)],
                    out_hbm.at[c, pl.ds(nbase, _NODE_ROWS_PER_TILE)])


def _scatter_body(src_hbm, dst_hbm, xs_hbm, zeros_hbm, out_hbm,
                  sidx, didx, b0, b1, acc_sh, gs0, gs1):
    c = lax.axis_index("c")
    s = lax.axis_index("s")
    nbase = s * _NODE_ROWS_PER_TILE
    pltpu.sync_copy(zeros_hbm.at[pl.ds(nbase, _NODE_ROWS_PER_TILE)],
                    acc_sh.at[pl.ds(nbase, _NODE_ROWS_PER_TILE)])
    row0 = (c * 16 + s) * _ROWS_PER_TILE
    pltpu.sync_copy(src_hbm.at[pl.ds(row0, _ROWS_PER_TILE)], sidx)
    pltpu.sync_copy(dst_hbm.at[pl.ds(row0, _ROWS_PER_TILE)], didx)
    plsc.subcore_barrier()

    bufs = (b0, b1)
    gsems = (gs0, gs1)

    def drain(slot):
        pltpu.make_async_copy(xs_hbm.at[pl.ds(0, 128)], bufs[slot],
                              gsems[slot]).wait()

    def step(ch, slot, issue_next):
        drain(slot)
        if issue_next:
            pltpu.async_copy(xs_hbm.at[sidx.at[ch + 1]], bufs[1 - slot],
                             gsems[1 - slot])
        pltpu.sync_copy(bufs[slot], acc_sh.at[didx.at[ch]], add=True)

    pltpu.async_copy(xs_hbm.at[sidx.at[0]], b0, gs0)

    def body(k, carry):
        step(2 * k, 0, True)
        step(2 * k + 1, 1, True)
        return carry

    lax.fori_loop(0, _ROWS_PER_TILE // 2 - 1, body, 0)
    step(_ROWS_PER_TILE - 2, 0, True)
    step(_ROWS_PER_TILE - 1, 1, False)
    plsc.subcore_barrier()
    pltpu.sync_copy(acc_sh.at[pl.ds(nbase, _NODE_ROWS_PER_TILE)],
                    out_hbm.at[c, pl.ds(nbase, _NODE_ROWS_PER_TILE)])


def _xs_body(x_ref, w_ref, dp_ref, o_ref):
    xw = jnp.dot(x_ref[...], w_ref[...], preferred_element_type=jnp.float32)
    deg = dp_ref[0] + dp_ref[1] + 1.0
    o_ref[...] = xw * lax.rsqrt(deg)


def _bn_body(acc_ref, xs_ref, dp_ref, b_ref, g_ref, be_ref, z_ref):
    deg = dp_ref[0] + dp_ref[1] + 1.0
    dinv = lax.rsqrt(deg)
    h = (acc_ref[0] + acc_ref[1] + xs_ref[...]) * dinv + b_ref[...]
    rowid = lax.broadcasted_iota(jnp.int32, (_NP, 1), 0)
    m = rowid < _N
    hm = jnp.where(m, h, 0.0)
    mean = jnp.sum(hm, axis=0, keepdims=True) * (1.0 / _N)
    ex2 = jnp.sum(hm * hm, axis=0, keepdims=True) * (1.0 / _N)
    var = ex2 - mean * mean
    rstd = lax.rsqrt(var + 1e-5)
    zn = (h - mean) * rstd * g_ref[...] + be_ref[...]
    z_ref[...] = jnp.where(m, jnp.maximum(zn, 0.0), 0.0)


def _dec_body(zi_ref, zj_ref, o_ref):
    zj = zj_ref[pl.ds(0, _N), :]
    o_ref[...] = lax.dot_general(
        zi_ref[...], zj, (((1,), (1,)), ((), ())),
        preferred_element_type=jnp.float32)


def kernel(x, edge_index, W, b, gamma, beta):
    f32 = jnp.float32
    src = edge_index[0]
    dst = edge_index[1]
    pad = _N + jnp.arange(_EP - _E, dtype=jnp.int32) % (_NP - _N)
    src2d = jnp.concatenate([src, pad]).reshape(_EROWS, 128)
    dst2d = jnp.concatenate([dst, pad]).reshape(_EROWS, 128)
    x_pad = jnp.zeros((_NP, _D), f32).at[:_N, :].set(x)
    zerosH = jnp.zeros((_NP, _H), f32)
    onesH = jnp.ones((128, _H), f32)

    mesh = plsc.VectorSubcoreMesh(core_axis_name="c", subcore_axis_name="s")

    deg_kernel = functools.partial(
        pl.kernel, mesh=mesh,
        out_type=jax.ShapeDtypeStruct((2, _NP, _H), f32),
        scratch_types=[
            pltpu.VMEM((128, _H), f32),
            pltpu.VMEM((_ROWS_PER_TILE, 128), jnp.int32),
            pltpu.VMEM_SHARED((_NP, _H), f32),
        ],
    )(_deg_body)
    degparts = deg_kernel(dst2d, zerosH, onesH)[:, :, 0:1]

    xs = pl.pallas_call(
        _xs_body,
        grid=(10,),
        in_specs=[
            pl.BlockSpec((1024, _D), lambda i: (i, 0)),
            pl.BlockSpec((_D, _H), lambda i: (0, 0)),
            pl.BlockSpec((2, 1024, 1), lambda i: (0, i, 0)),
        ],
        out_specs=pl.BlockSpec((1024, _H), lambda i: (i, 0)),
        out_shape=jax.ShapeDtypeStruct((_NP, _H), f32),
    )(x_pad, W, degparts)

    scatter_kernel = functools.partial(
        pl.kernel, mesh=mesh,
        out_type=jax.ShapeDtypeStruct((2, _NP, _H), f32),
        scratch_types=[
            pltpu.VMEM((_ROWS_PER_TILE, 128), jnp.int32),
            pltpu.VMEM((_ROWS_PER_TILE, 128), jnp.int32),
            pltpu.VMEM((128, _H), f32),
            pltpu.VMEM((128, _H), f32),
            pltpu.VMEM_SHARED((_NP, _H), f32),
            pltpu.SemaphoreType.DMA,
            pltpu.SemaphoreType.DMA,
        ],
    )(_scatter_body)
    accparts = scatter_kernel(src2d, dst2d, xs, zerosH)

    z = pl.pallas_call(
        _bn_body,
        in_specs=[
            pl.BlockSpec((2, _NP, _H), lambda: (0, 0, 0)),
            pl.BlockSpec((_NP, _H), lambda: (0, 0)),
            pl.BlockSpec((2, _NP, 1), lambda: (0, 0, 0)),
            pl.BlockSpec((_H,), lambda: (0,)),
            pl.BlockSpec((_H,), lambda: (0,)),
            pl.BlockSpec((_H,), lambda: (0,)),
        ],
        out_specs=pl.BlockSpec((_NP, _H), lambda: (0, 0)),
        out_shape=jax.ShapeDtypeStruct((_NP, _H), f32),
    )(accparts, xs, degparts, b, gamma, beta)

    adj = pl.pallas_call(
        _dec_body,
        grid=(50,),
        in_specs=[
            pl.BlockSpec((200, _H), lambda i: (i, 0)),
            pl.BlockSpec((_NP, _H), lambda i: (0, 0)),
        ],
        out_specs=pl.BlockSpec((200, _N), lambda i: (i, 0)),
        out_shape=jax.ShapeDtypeStruct((_N, _N), f32),
    )(z, z)
    return adj
